# Initial kernel scaffold; baseline (speedup 1.0000x reference)
import jax
import jax.numpy as jnp
from jax.experimental import pallas as pl


def kernel(pred, labels, area):
    raise NotImplementedError("write your pallas kernel here")


# speedup vs baseline: 86.1614x; 86.1614x over previous
import functools

import jax
import jax.numpy as jnp
from jax import lax
from jax.experimental import pallas as pl

_SMOOTH = 0.001
_NEG_BIG = -1e30
_BIG = 1e30


def _corners4(x):
    x01 = jnp.roll(x, -1, axis=1)
    x10 = jnp.roll(x, -1, axis=0)
    x11 = jnp.roll(x10, -1, axis=1)
    return [x, x01, x10, x11]


def _dotc(a, b):
    a2 = a.reshape(16, -1)
    b2 = b.reshape(16, -1)
    return lax.dot_general(a2, b2, (((1,), (1,)), ((), ())),
                           preferred_element_type=jnp.float32)


def _hist_kernel(p0_ref, p1_ref, l0_ref, l1_ref, out_ref, *, valid_r, valid_c):
    zp = pl.program_id(1)
    f32 = jnp.float32

    s0 = jax.nn.sigmoid(p0_ref[0, 0])
    s1 = jax.nn.sigmoid(p1_ref[0, 0])
    R, C = s0.shape

    cp = _corners4(s0) + _corners4(s1)
    lb = [v > 0.5 for v in _corners4(l0_ref[0, 0]) + _corners4(l1_ref[0, 0])]

    rows = lax.broadcasted_iota(jnp.int32, (R, C), 0)
    cols = lax.broadcasted_iota(jnp.int32, (R, C), 1)
    valid = (rows < valid_r) & (cols < valid_c)

    i16 = lax.broadcasted_iota(jnp.int32, (16, R, C), 0).astype(f32)

    lf = [jnp.where(b, f32(1.0), f32(0.0)) for b in lb]
    llo = lf[0] + 2.0 * lf[1] + 4.0 * lf[2] + 8.0 * lf[3]
    lhi = lf[4] + 2.0 * lf[5] + 4.0 * lf[6] + 8.0 * lf[7]
    oh_llo = (llo[None] == i16).astype(f32)
    oh_lhi = lhi[None] == i16

    pw = jnp.zeros((R, C), f32)
    hp = jnp.zeros((16, 16), f32)

    for _ in range(8):
        nzb = [c > 0 for c in cp]
        nzf = [jnp.where(b, f32(1.0), f32(0.0)) for b in nzb]
        masked = [jnp.where(b, c, f32(_BIG)) for b, c in zip(nzb, cp)]
        sw = jnp.minimum(
            jnp.minimum(jnp.minimum(masked[0], masked[1]),
                        jnp.minimum(masked[2], masked[3])),
            jnp.minimum(jnp.minimum(masked[4], masked[5]),
                        jnp.minimum(masked[6], masked[7])))
        sw = jnp.where(sw < f32(1e29), sw, f32(0.0))

        eq = [nzb[k] == lb[k] for k in range(8)]
        match = ((eq[0] & eq[1]) & (eq[2] & eq[3])) & \
                ((eq[4] & eq[5]) & (eq[6] & eq[7]))
        pw = pw + jnp.where(match, sw, f32(0.0))

        lo = nzf[0] + 2.0 * nzf[1] + 4.0 * nzf[2] + 8.0 * nzf[3]
        hi = nzf[4] + 2.0 * nzf[5] + 4.0 * nzf[6] + 8.0 * nzf[7]
        x = (hi[None] == i16).astype(f32) * sw[None]
        y = (lo[None] == i16).astype(f32)
        hp = hp + _dotc(x, y)

        cp = [c - sw * f for c, f in zip(cp, nzf)]

    xv = (oh_lhi & valid[None]).astype(f32)
    xpw = jnp.where(oh_lhi, pw[None], f32(0.0))
    hl = _dotc(xv, oh_llo)
    hi_ = _dotc(xpw, oh_llo)

    @pl.when(zp == 0)
    def _init():
        out_ref[...] = jnp.zeros_like(out_ref)

    out_ref[0, 0] = out_ref[0, 0] + hp
    out_ref[0, 1] = out_ref[0, 1] + hl
    out_ref[0, 2] = out_ref[0, 2] + hi_


def kernel(pred, labels, area):
    B, Z, H, W = pred.shape
    Rp = ((H + 2 + 7) // 8) * 8
    Cp = ((W + 2 + 127) // 128) * 128

    f32 = jnp.float32
    pad_p = jnp.full((B, Z, Rp, Cp), _NEG_BIG, f32)
    pad_p = pad_p.at[:, :, 1:H + 1, 1:W + 1].set(pred.astype(f32))
    pad_l = jnp.zeros((B, Z, Rp, Cp), f32)
    pad_l = pad_l.at[:, :, 1:H + 1, 1:W + 1].set(labels.astype(f32))

    grid = (B, Z - 1)
    blk = (1, 1, Rp, Cp)
    hists = pl.pallas_call(
        functools.partial(_hist_kernel, valid_r=H + 1, valid_c=W + 1),
        grid=grid,
        in_specs=[
            pl.BlockSpec(blk, lambda b, z: (b, z, 0, 0)),
            pl.BlockSpec(blk, lambda b, z: (b, z + 1, 0, 0)),
            pl.BlockSpec(blk, lambda b, z: (b, z, 0, 0)),
            pl.BlockSpec(blk, lambda b, z: (b, z + 1, 0, 0)),
        ],
        out_specs=pl.BlockSpec((1, 3, 16, 16), lambda b, z: (b, 0, 0, 0)),
        out_shape=jax.ShapeDtypeStruct((B, 3, 16, 16), f32),
    )(pad_p, pad_p, pad_l, pad_l)

    a16 = area.astype(f32).reshape(16, 16)
    denom = ((hists[:, 0] + hists[:, 1]) * a16[None]).sum(axis=(1, 2))
    num = 2.0 * (hists[:, 2] * a16[None]).sum(axis=(1, 2))
    dice = 1.0 - (num + _SMOOTH) / (denom + _SMOOTH)
    return dice.mean()


# speedup vs baseline: 206.5308x; 2.3970x over previous
import functools

import jax
import jax.numpy as jnp
from jax import lax
from jax.experimental import pallas as pl

_SMOOTH = 0.001
_NEG_BIG = -1e30
_BIG = 1e30


def _corners4(x):
    x01 = jnp.roll(x, -1, axis=1)
    x10 = jnp.roll(x, -1, axis=0)
    x11 = jnp.roll(x10, -1, axis=1)
    return [x, x01, x10, x11]


def _dotc(a, b):
    return lax.dot_general(a, b, (((1,), (1,)), ((), ())),
                           preferred_element_type=jnp.float32)


def _hist_kernel(p0_ref, p1_ref, l0_ref, l1_ref, out_ref, *, valid_r, valid_c):
    zp = pl.program_id(1)
    f32 = jnp.float32
    bf16 = jnp.bfloat16

    s0 = jax.nn.sigmoid(p0_ref[0, 0])
    s1 = jax.nn.sigmoid(p1_ref[0, 0])
    R, C = s0.shape
    N = R * C

    cp = _corners4(s0) + _corners4(s1)

    la = _corners4(l0_ref[0, 0])
    lb_ = _corners4(l1_ref[0, 0])
    llo = la[0] + 2.0 * la[1] + 4.0 * la[2] + 8.0 * la[3]
    lhi = lb_[0] + 2.0 * lb_[1] + 4.0 * lb_[2] + 8.0 * lb_[3]

    rowv = (lax.broadcasted_iota(jnp.int32, (R, 1), 0) < valid_r).astype(bf16)
    colv = (lax.broadcasted_iota(jnp.int32, (1, C), 1) < valid_c).astype(bf16)

    i16 = lax.broadcasted_iota(jnp.int32, (16, N), 0).astype(bf16)
    zero16 = jnp.zeros((16, N), bf16)

    llo_f = llo.astype(bf16).reshape(1, N)
    lhi_f = lhi.astype(bf16).reshape(1, N)
    oh_llo = jnp.where(llo_f == i16, bf16(1.0), bf16(0.0))
    oh_lhi_b = lhi_f == i16

    pw = jnp.zeros((R, C), f32)
    hp = jnp.zeros((16, 16), f32)

    for _ in range(8):
        nzb = [c > 0 for c in cp]
        nzf = [jnp.where(b, f32(1.0), f32(0.0)) for b in nzb]
        masked = [jnp.where(b, c, f32(_BIG)) for b, c in zip(nzb, cp)]
        sw = jnp.minimum(
            jnp.minimum(jnp.minimum(masked[0], masked[1]),
                        jnp.minimum(masked[2], masked[3])),
            jnp.minimum(jnp.minimum(masked[4], masked[5]),
                        jnp.minimum(masked[6], masked[7])))
        sw = jnp.where(sw < f32(1e29), sw, f32(0.0))

        lo = nzf[0] + 2.0 * nzf[1] + 4.0 * nzf[2] + 8.0 * nzf[3]
        hi = nzf[4] + 2.0 * nzf[5] + 4.0 * nzf[6] + 8.0 * nzf[7]

        match = (lo == llo) & (hi == lhi)
        pw = pw + jnp.where(match, sw, f32(0.0))

        lo_f = lo.astype(bf16).reshape(1, N)
        hi_f = hi.astype(bf16).reshape(1, N)
        sw_f = sw.astype(bf16).reshape(1, N)
        x = jnp.where(hi_f == i16, sw_f, zero16)
        y = jnp.where(lo_f == i16, bf16(1.0), bf16(0.0))
        hp = hp + _dotc(x, y)

        cp = [c - sw * f for c, f in zip(cp, nzf)]

    valid_f = (rowv * colv).reshape(1, N)
    pw_f = pw.astype(bf16).reshape(1, N)
    xv = jnp.where(oh_lhi_b, valid_f, zero16)
    xpw = jnp.where(oh_lhi_b, pw_f, zero16)
    hl = _dotc(xv, oh_llo)
    hi_ = _dotc(xpw, oh_llo)

    @pl.when(zp == 0)
    def _init():
        out_ref[...] = jnp.zeros_like(out_ref)

    out_ref[0, 0] = out_ref[0, 0] + hp
    out_ref[0, 1] = out_ref[0, 1] + hl
    out_ref[0, 2] = out_ref[0, 2] + hi_


def kernel(pred, labels, area):
    B, Z, H, W = pred.shape
    Rp = ((H + 2 + 7) // 8) * 8
    Cp = ((W + 2 + 127) // 128) * 128

    f32 = jnp.float32
    pad_p = jnp.full((B, Z, Rp, Cp), _NEG_BIG, f32)
    pad_p = pad_p.at[:, :, 1:H + 1, 1:W + 1].set(pred.astype(f32))
    pad_l = jnp.zeros((B, Z, Rp, Cp), f32)
    pad_l = pad_l.at[:, :, 1:H + 1, 1:W + 1].set(labels.astype(f32))

    grid = (B, Z - 1)
    blk = (1, 1, Rp, Cp)
    hists = pl.pallas_call(
        functools.partial(_hist_kernel, valid_r=H + 1, valid_c=W + 1),
        grid=grid,
        in_specs=[
            pl.BlockSpec(blk, lambda b, z: (b, z, 0, 0)),
            pl.BlockSpec(blk, lambda b, z: (b, z + 1, 0, 0)),
            pl.BlockSpec(blk, lambda b, z: (b, z, 0, 0)),
            pl.BlockSpec(blk, lambda b, z: (b, z + 1, 0, 0)),
        ],
        out_specs=pl.BlockSpec((1, 3, 16, 16), lambda b, z: (b, 0, 0, 0)),
        out_shape=jax.ShapeDtypeStruct((B, 3, 16, 16), f32),
    )(pad_p, pad_p, pad_l, pad_l)

    a16 = area.astype(f32).reshape(16, 16)
    denom = ((hists[:, 0] + hists[:, 1]) * a16[None]).sum(axis=(1, 2))
    num = 2.0 * (hists[:, 2] * a16[None]).sum(axis=(1, 2))
    dice = 1.0 - (num + _SMOOTH) / (denom + _SMOOTH)
    return dice.mean()
